# baseline (speedup 1.0000x reference)
import functools

import jax
import jax.numpy as jnp
from jax import lax
from jax.experimental import pallas as pl
from jax.experimental.pallas import tpu as pltpu
from jax.experimental.pallas import tpu_sc as plsc

_LANES = 16
_RING = 12


@functools.lru_cache(maxsize=None)
def _build(V, D, B):
    info = plsc.get_sparse_core_info()
    NC, NS = info.num_cores, info.num_subcores
    NW = NC * NS
    b_per_w = B // NW
    quart = b_per_w // 4
    mesh = plsc.VectorSubcoreMesh(core_axis_name="c", subcore_axis_name="s")
    n_chunk = D // _LANES
    n_grp = (b_per_w - 8) // _RING
    tail = b_per_w - n_grp * _RING

    @functools.partial(
        pl.kernel,
        mesh=mesh,
        out_type=jax.ShapeDtypeStruct((2, D, B), jnp.float32),
        scratch_types=[
            pltpu.SMEM((2, b_per_w), jnp.int32),
            pltpu.VMEM((2 * b_per_w,), jnp.int32),
            pltpu.VMEM((_RING, D, 128), jnp.float32),
            pltpu.VMEM((2, D, quart), jnp.float32),
            pltpu.SemaphoreType.DMA((_RING,)),
            pltpu.SemaphoreType.DMA,
        ],
        compiler_params=pltpu.CompilerParams(needs_layout_passes=False),
    )
    def k(wtT_hbm, ctT_hbm, words_hbm, ctx_hbm, out_hbm, idx_s, idx_v,
          blk_v, rows_v, ring_sem, out_sem):
        wid = lax.axis_index("s") * NC + lax.axis_index("c")
        base = wid * b_per_w

        pltpu.sync_copy(words_hbm.at[pl.ds(base, b_per_w)],
                        idx_v.at[pl.ds(0, b_per_w)])
        pltpu.sync_copy(ctx_hbm.at[pl.ds(base, b_per_w)],
                        idx_v.at[pl.ds(b_per_w, b_per_w)])

        tables = (wtT_hbm, ctT_hbm)
        lane = lax.iota(jnp.int32, _LANES)

        for t in range(2):
            def stage(g, t=t):
                vec = plsc.load_gather(
                    idx_v, [t * b_per_w + g * _LANES + lane]
                )
                for l in range(_LANES):
                    s = lax.reduce_max(
                        jnp.where(lane == l, vec, jnp.int32(-2147483648)),
                        axes=(0,),
                    )
                    idx_s[t, g * _LANES + l] = s
            pl.loop(0, b_per_w // _LANES)(stage)

        def fire(t, j, slot):
            i = jnp.clip(idx_s[t, j], 0, V - 1)
            blk = pl.multiple_of((i >> 7) * 128, 128)
            return pltpu.async_copy(
                tables[t].at[:, pl.ds(blk, 128)],
                blk_v.at[slot],
                ring_sem.at[slot],
            )

        def wait_slot(t, slot):
            pltpu.make_async_copy(
                tables[t].at[:, pl.ds(0, 128)],
                blk_v.at[slot],
                ring_sem.at[slot],
            ).wait()

        def extract(t, j, slot):
            i = idx_s[t, j]
            col = jnp.full((_LANES,), i & 127, dtype=jnp.int32)
            hvec = jnp.full((_LANES,), (j >> 7) & 1, dtype=jnp.int32)
            cvec = jnp.full((_LANES,), j & (quart - 1), dtype=jnp.int32)
            for c in range(n_chunk):
                dvec = lane + c * _LANES
                vals = plsc.load_gather(blk_v.at[slot], [dvec, col])
                plsc.store_scatter(rows_v, [hvec, dvec, cvec], vals)

        def flush(t, q, buf):
            return pltpu.async_copy(
                rows_v.at[buf],
                out_hbm.at[t, :, pl.ds(base + q * quart, quart)],
                out_sem,
            )

        def wait_flush(t, q, buf):
            pltpu.make_async_copy(
                rows_v.at[buf],
                out_hbm.at[t, :, pl.ds(base + q * quart, quart)],
                out_sem,
            ).wait()

        for t in range(2):
            for r in range(_RING):
                fire(t, r, r)

            def grp(g, t=t):
                for r in range(_RING):
                    j = g * _RING + r
                    wait_slot(t, r)
                    extract(t, j, r)

                    @pl.when(j < b_per_w - _RING)
                    def _():
                        fire(t, j + _RING, r)

                    @pl.when(j == quart - 1)
                    def _():
                        flush(t, 0, 0)

                    @pl.when(j == 2 * quart - 1)
                    def _():
                        flush(t, 1, 1)
                        wait_flush(t, 0, 0)

                    @pl.when(j == 3 * quart - 1)
                    def _():
                        flush(t, 2, 0)
                        wait_flush(t, 1, 1)

            pl.loop(0, n_grp)(grp)

            for r in range(tail):
                j = n_grp * _RING + r
                wait_slot(t, r)
                extract(t, j, r)
            flush(t, 3, 1)
            if t == 0:
                wait_flush(0, 2, 0)
                wait_flush(0, 3, 1)
        wait_flush(1, 2, 0)
        wait_flush(1, 3, 1)

    return k


def kernel(words, contexts, w_table, c_table):
    V, D = w_table.shape
    B = words.shape[0]
    k = _build(V, D, B)
    out_t = k(
        jnp.swapaxes(w_table, 0, 1),
        jnp.swapaxes(c_table, 0, 1),
        words,
        contexts,
    )
    return jnp.swapaxes(out_t, 1, 2)
